# baseline (speedup 1.0000x reference)
import jax
import jax.numpy as jnp
from jax.experimental import pallas as pl
from jax.experimental.pallas import tpu as pltpu

_BS = 4
_MAX_SEQLEN = 128
_LAYERS = 6
_HEADS = 32
_HEAD_DIM = 128
_STRIDE = 16
_NUM_PAGES = 64
_FEAT = _HEADS * _HEAD_DIM
_ROWS = _LAYERS * 2 * _STRIDE
_H_ROWS = _BS * _MAX_SEQLEN
_H_CHUNK = 64
_H_N = _H_ROWS // _H_CHUNK


def _layer_consts():
    x = jnp.float32(1.0)
    ks, vs = [], []
    for _ in range(_LAYERS):
        xk = x * jnp.float32(2.0)
        xv = x * jnp.float32(4.0)
        ks.append(xk)
        vs.append(xv)
        x = x + x * xk * xv
    return ks, vs, x


def _body(mask_ref, in_hbm, out_hbm, h_hbm, pat_vmem, h_vmem, sem, hsem):
    ks, vs, h_final = _layer_consts()

    r = jax.lax.broadcasted_iota(jnp.int32, (_ROWS, _FEAT), 0)
    layer_idx = r // (2 * _STRIDE)
    kv = (r // _STRIDE) % 2
    pat = jnp.zeros((_ROWS, _FEAT), jnp.float32)
    for l in range(_LAYERS):
        pat = jnp.where(layer_idx == l, jnp.where(kv == 0, ks[l], vs[l]), pat)
    pat_vmem[...] = pat
    h_vmem[...] = jnp.full((_H_CHUNK, _FEAT), h_final)

    for p in range(_NUM_PAGES):
        member = mask_ref[p] > 0

        @pl.when(member)
        def _():
            pltpu.make_async_copy(pat_vmem, out_hbm.at[p], sem).start()

        @pl.when(jnp.logical_not(member))
        def _():
            pltpu.make_async_copy(in_hbm.at[p], out_hbm.at[p], sem).start()

    for i in range(_H_N):
        pltpu.make_async_copy(
            h_vmem, h_hbm.at[pl.ds(i * _H_CHUNK, _H_CHUNK)], hsem).start()

    for p in range(_NUM_PAGES):
        pltpu.make_async_copy(in_hbm.at[p], out_hbm.at[p], sem).wait()
    for i in range(_H_N):
        pltpu.make_async_copy(
            h_vmem, h_hbm.at[pl.ds(i * _H_CHUNK, _H_CHUNK)], hsem).wait()


def kernel(seq_lens, attn_block_ids, attn_page_slab):
    del seq_lens
    ids = attn_block_ids.reshape(-1).astype(jnp.int32)
    mask = jnp.zeros((_NUM_PAGES,), jnp.int32).at[ids].set(1)
    slab = attn_page_slab.reshape(_NUM_PAGES, _ROWS, _FEAT)

    out, h = pl.pallas_call(
        _body,
        in_specs=[
            pl.BlockSpec(memory_space=pltpu.SMEM),
            pl.BlockSpec(memory_space=pl.ANY),
        ],
        out_specs=[
            pl.BlockSpec(memory_space=pl.ANY),
            pl.BlockSpec(memory_space=pl.ANY),
        ],
        out_shape=[
            jax.ShapeDtypeStruct((_NUM_PAGES, _ROWS, _FEAT), jnp.float32),
            jax.ShapeDtypeStruct((_H_ROWS, _FEAT), jnp.float32),
        ],
        scratch_shapes=[
            pltpu.VMEM((_ROWS, _FEAT), jnp.float32),
            pltpu.VMEM((_H_CHUNK, _FEAT), jnp.float32),
            pltpu.SemaphoreType.DMA,
            pltpu.SemaphoreType.DMA,
        ],
    )(mask, slab)

    h = h.reshape(_BS, _MAX_SEQLEN, _FEAT)
    slab_out = out.reshape(_NUM_PAGES, _LAYERS, 2, _STRIDE, _HEADS, _HEAD_DIM)
    return h, slab_out


# speedup vs baseline: 7.3907x; 7.3907x over previous
import jax
import jax.numpy as jnp
from jax.experimental import pallas as pl
from jax.experimental.pallas import tpu as pltpu

_BS = 4
_MAX_SEQLEN = 128
_LAYERS = 6
_HEADS = 32
_HEAD_DIM = 128
_STRIDE = 16
_NUM_PAGES = 64
_FEAT = _HEADS * _HEAD_DIM
_ROWS = _LAYERS * 2 * _STRIDE
_H_ROWS = _BS * _MAX_SEQLEN
_H_BLOCK = _H_ROWS // _NUM_PAGES


def _layer_consts():
    x = jnp.float32(1.0)
    ks, vs = [], []
    for _ in range(_LAYERS):
        xk = x * jnp.float32(2.0)
        xv = x * jnp.float32(4.0)
        ks.append(xk)
        vs.append(xv)
        x = x + x * xk * xv
    return ks, vs, x


def _body(mask_ref, src_ref, in_ref, out_ref, h_ref, pat_ref):
    p = pl.program_id(0)
    ks, vs, h_final = _layer_consts()

    @pl.when(p == 0)
    def _():
        r = jax.lax.broadcasted_iota(jnp.int32, (_ROWS, _FEAT), 0)
        layer_idx = r // (2 * _STRIDE)
        kv = (r // _STRIDE) % 2
        pat = jnp.zeros((_ROWS, _FEAT), jnp.float32)
        for l in range(_LAYERS):
            pat = jnp.where(layer_idx == l, jnp.where(kv == 0, ks[l], vs[l]), pat)
        pat_ref[...] = pat

    member = mask_ref[p] > 0

    @pl.when(member)
    def _():
        out_ref[...] = pat_ref[...][None]

    @pl.when(jnp.logical_not(member))
    def _():
        out_ref[...] = in_ref[...]

    h_ref[...] = jnp.full((_H_BLOCK, _FEAT), h_final)


def kernel(seq_lens, attn_block_ids, attn_page_slab):
    del seq_lens
    ids = attn_block_ids.reshape(-1).astype(jnp.int32)
    mask = jnp.zeros((_NUM_PAGES,), jnp.int32).at[ids].set(1)
    fallback = jnp.argmin(mask).astype(jnp.int32)
    src = jnp.where(mask > 0, fallback, jnp.arange(_NUM_PAGES, dtype=jnp.int32))
    slab = attn_page_slab.reshape(_NUM_PAGES, _ROWS, _FEAT)

    grid_spec = pltpu.PrefetchScalarGridSpec(
        num_scalar_prefetch=2,
        grid=(_NUM_PAGES,),
        in_specs=[
            pl.BlockSpec((1, _ROWS, _FEAT), lambda i, mask, src: (src[i], 0, 0)),
        ],
        out_specs=[
            pl.BlockSpec((1, _ROWS, _FEAT), lambda i, mask, src: (i, 0, 0)),
            pl.BlockSpec((_H_BLOCK, _FEAT), lambda i, mask, src: (i, 0)),
        ],
        scratch_shapes=[
            pltpu.VMEM((_ROWS, _FEAT), jnp.float32),
        ],
    )

    out, h = pl.pallas_call(
        _body,
        grid_spec=grid_spec,
        out_shape=[
            jax.ShapeDtypeStruct((_NUM_PAGES, _ROWS, _FEAT), jnp.float32),
            jax.ShapeDtypeStruct((_H_ROWS, _FEAT), jnp.float32),
        ],
        compiler_params=pltpu.CompilerParams(
            dimension_semantics=("arbitrary",),
        ),
    )(mask, src, slab)

    h = h.reshape(_BS, _MAX_SEQLEN, _FEAT)
    slab_out = out.reshape(_NUM_PAGES, _LAYERS, 2, _STRIDE, _HEADS, _HEAD_DIM)
    return h, slab_out


# speedup vs baseline: 32.8457x; 4.4442x over previous
import jax
import jax.numpy as jnp
from jax.experimental import pallas as pl
from jax.experimental.pallas import tpu as pltpu

_BS = 4
_MAX_SEQLEN = 128
_LAYERS = 6
_HEADS = 32
_HEAD_DIM = 128
_STRIDE = 16
_NUM_PAGES = 64
_FEAT = _HEADS * _HEAD_DIM
_PAGE = (1, _LAYERS, 2, _STRIDE, _HEADS, _HEAD_DIM)
_H_SPLIT = _NUM_PAGES // _BS
_H_BLOCK = _MAX_SEQLEN // _H_SPLIT


def _layer_consts():
    x = jnp.float32(1.0)
    ks, vs = [], []
    for _ in range(_LAYERS):
        xk = x * jnp.float32(2.0)
        xv = x * jnp.float32(4.0)
        ks.append(xk)
        vs.append(xv)
        x = x + x * xk * xv
    return ks, vs, x


def _body(mask_ref, src_ref, in_ref, out_ref, h_ref, pat_ref):
    p = pl.program_id(0)
    ks, vs, h_final = _layer_consts()

    @pl.when(p == 0)
    def _():
        l_idx = jax.lax.broadcasted_iota(jnp.int32, _PAGE, 1)
        kv_idx = jax.lax.broadcasted_iota(jnp.int32, _PAGE, 2)
        pat = jnp.zeros(_PAGE, jnp.float32)
        for l in range(_LAYERS):
            pat = jnp.where(l_idx == l,
                            jnp.where(kv_idx == 0, ks[l], vs[l]), pat)
        pat_ref[...] = pat

    member = mask_ref[p] > 0

    @pl.when(member)
    def _():
        out_ref[...] = pat_ref[...]

    @pl.when(jnp.logical_not(member))
    def _():
        out_ref[...] = in_ref[...]

    h_ref[...] = jnp.full((1, _H_BLOCK, _FEAT), h_final)


def kernel(seq_lens, attn_block_ids, attn_page_slab):
    del seq_lens
    ids = attn_block_ids.reshape(-1).astype(jnp.int32)
    mask = jnp.zeros((_NUM_PAGES,), jnp.int32).at[ids].set(1)
    fallback = jnp.argmin(mask).astype(jnp.int32)
    src = jnp.where(mask > 0, fallback, jnp.arange(_NUM_PAGES, dtype=jnp.int32))

    grid_spec = pltpu.PrefetchScalarGridSpec(
        num_scalar_prefetch=2,
        grid=(_NUM_PAGES,),
        in_specs=[
            pl.BlockSpec(_PAGE, lambda i, mask, src: (src[i], 0, 0, 0, 0, 0)),
        ],
        out_specs=[
            pl.BlockSpec(_PAGE, lambda i, mask, src: (i, 0, 0, 0, 0, 0)),
            pl.BlockSpec((1, _H_BLOCK, _FEAT),
                         lambda i, mask, src: (i // _H_SPLIT, i % _H_SPLIT, 0)),
        ],
        scratch_shapes=[
            pltpu.VMEM(_PAGE, jnp.float32),
        ],
    )

    out, h = pl.pallas_call(
        _body,
        grid_spec=grid_spec,
        out_shape=[
            jax.ShapeDtypeStruct(
                (_NUM_PAGES, _LAYERS, 2, _STRIDE, _HEADS, _HEAD_DIM),
                jnp.float32),
            jax.ShapeDtypeStruct((_BS, _MAX_SEQLEN, _FEAT), jnp.float32),
        ],
        compiler_params=pltpu.CompilerParams(
            dimension_semantics=("arbitrary",),
        ),
    )(mask, src, attn_page_slab)

    return h, out
